# baseline (speedup 1.0000x reference)
import functools
import jax
import jax.numpy as jnp
from jax import lax
from jax.experimental import pallas as pl
from jax.experimental.pallas import tpu as pltpu
from jax.experimental.pallas import tpu_sc as plsc

_N, _D, _H, _E = 2048, 1024, 512, 8
_BS = 128
_MPAD = 4096 + _E * _BS
_NB = _MPAD // _BS
_NCH = 16
_CH = _N // _NCH

_NC, _NS = 2, 16
_NW = _NC * _NS
_TPW = _N // _NW
_TCH = 16
_L = 16
_WL = 128


def _route_kernel(x_ref, wg_ref, pos0_ref, pos1_ref, w0_ref, w1_ref, be_ref):
    x = x_ref[...]
    wg = wg_ref[...]
    logits = lax.dot_general(x, wg, (((1,), (1,)), ((), ())),
                             preferred_element_type=jnp.float32)
    m = jnp.max(logits, axis=-1, keepdims=True)
    p = jnp.exp(logits - m)
    lane = lax.broadcasted_iota(jnp.int32, p.shape, 1)
    p0 = jnp.max(p, axis=-1, keepdims=True)
    e0 = jnp.min(jnp.where(p == p0, lane, _E), axis=-1, keepdims=True)
    p_m = jnp.where(lane == e0, -jnp.inf, p)
    p1 = jnp.max(p_m, axis=-1, keepdims=True)
    e1 = jnp.min(jnp.where(p_m == p1, lane, _E), axis=-1, keepdims=True)
    s = p0 + p1
    w0_ref[...] = jnp.broadcast_to(p0 / s, (_N, _WL))
    w1_ref[...] = jnp.broadcast_to(p1 / s, (_N, _WL))

    oh0 = (lane == e0).astype(jnp.float32)
    oh1 = (lane == e1).astype(jnp.float32)
    cnt = oh0 + oh1

    a = jnp.concatenate([cnt[c * _CH:(c + 1) * _CH, :] for c in range(_NCH)],
                        axis=1)
    ri = lax.broadcasted_iota(jnp.int32, (_CH, _CH), 0)
    ci = lax.broadcasted_iota(jnp.int32, (_CH, _CH), 1)
    ltri = (ri >= ci).astype(jnp.float32)
    b = lax.dot_general(ltri, a, (((1,), (0,)), ((), ())),
                        preferred_element_type=jnp.float32)
    tot = b[_CH - 1:_CH, :]
    same_e = (ri % _E) == (ci % _E)
    before = (ri // _E) < (ci // _E)
    mcarry = jnp.where(jnp.logical_and(same_e, before), 1.0, 0.0)
    carry = lax.dot_general(tot, mcarry, (((1,), (0,)), ((), ())),
                            preferred_element_type=jnp.float32)
    bc = b + carry
    csum = jnp.concatenate([bc[:, c * _E:(c + 1) * _E] for c in range(_NCH)],
                           axis=0)

    cnt_tot = csum[_N - 1:_N, :]
    cnt_pad = jnp.floor((cnt_tot + (_BS - 1)) * (1.0 / _BS)) * _BS
    ei = lax.broadcasted_iota(jnp.int32, (_E, _E), 0)
    ej = lax.broadcasted_iota(jnp.int32, (_E, _E), 1)
    excl = jnp.where(ei < ej, 1.0, 0.0)
    off = lax.dot_general(cnt_pad, excl, (((1,), (0,)), ((), ())),
                          preferred_element_type=jnp.float32)

    off_b = jnp.broadcast_to(off, (_N, _E))
    rank0 = jnp.sum(jnp.where(lane == e0, csum, 0.0), -1, keepdims=True) - 1.0
    rank1 = jnp.sum(jnp.where(lane == e1, csum, 0.0), -1, keepdims=True) - 1.0
    base0 = jnp.sum(jnp.where(lane == e0, off_b, 0.0), -1, keepdims=True)
    base1 = jnp.sum(jnp.where(lane == e1, off_b, 0.0), -1, keepdims=True)
    pos0_ref[...] = (base0 + rank0).astype(jnp.int32)
    pos1_ref[...] = (base1 + rank1).astype(jnp.int32)

    blk = lax.broadcasted_iota(jnp.int32, (1, 128), 1).astype(jnp.float32) * float(_BS)
    be = jnp.zeros((1, 128), jnp.float32)
    for e in range(_E):
        lo = off[:, e:e + 1]
        hi = lo + cnt_pad[:, e:e + 1]
        ind = jnp.where(jnp.logical_and(blk >= lo, blk < hi), 1.0, 0.0)
        be = be + float(e) * ind
    be_ref[...] = be.astype(jnp.int32)


def _route(x, Wg):
    return pl.pallas_call(
        _route_kernel,
        out_shape=[
            jax.ShapeDtypeStruct((_N, 1), jnp.int32),
            jax.ShapeDtypeStruct((_N, 1), jnp.int32),
            jax.ShapeDtypeStruct((_N, _WL), jnp.float32),
            jax.ShapeDtypeStruct((_N, _WL), jnp.float32),
            jax.ShapeDtypeStruct((1, 128), jnp.int32),
        ],
    )(x, Wg)


def _dispatch(xb, pos0f, pos1f, w0s, w1s):
    mesh = plsc.VectorSubcoreMesh(core_axis_name="c", subcore_axis_name="s")

    @functools.partial(
        pl.kernel, mesh=mesh,
        out_type=[
            jax.ShapeDtypeStruct((_MPAD, _D), jnp.float32),
            jax.ShapeDtypeStruct((_MPAD, _WL), jnp.float32),
        ],
        scratch_types=[
            pltpu.VMEM((_TPW,), jnp.int32),
            pltpu.VMEM((_TPW,), jnp.int32),
            pltpu.VMEM((_TPW, _D), jnp.float32),
            pltpu.VMEM((_TPW, _WL), jnp.float32),
            pltpu.VMEM((_TPW, _WL), jnp.float32),
            pltpu.SemaphoreType.DMA,
            pltpu.SemaphoreType.DMA,
        ],
    )
    def dispatch_k(x_hbm, pos0_hbm, pos1_hbm, w0_hbm, w1_hbm, xs_hbm, ws_hbm,
                   i0_v, i1_v, rows_v, w0_v, w1_v, sem, semw):
        wid = lax.axis_index("s") * _NC + lax.axis_index("c")
        base = wid * _TPW
        pltpu.sync_copy(pos0_hbm.at[pl.ds(base, _TPW)], i0_v)
        pltpu.sync_copy(pos1_hbm.at[pl.ds(base, _TPW)], i1_v)
        pltpu.sync_copy(x_hbm.at[pl.ds(base, _TPW)], rows_v)
        pltpu.sync_copy(w0_hbm.at[pl.ds(base, _TPW)], w0_v)
        pltpu.sync_copy(w1_hbm.at[pl.ds(base, _TPW)], w1_v)
        cw0 = pltpu.async_copy(w0_v, ws_hbm.at[i0_v], semw)
        cw1 = pltpu.async_copy(w1_v, ws_hbm.at[i1_v], semw)
        cx0 = pltpu.async_copy(rows_v, xs_hbm.at[i0_v], sem)
        cx1 = pltpu.async_copy(rows_v, xs_hbm.at[i1_v], sem)
        cw0.wait()
        cw1.wait()
        cx0.wait()
        cx1.wait()

    return dispatch_k(xb, pos0f, pos1f, w0s, w1s)


def _ffn_kernel(be_ref, xs_ref, ws_ref, w1_ref, w2_ref, lnw_ref, lnb_ref,
                ys_ref):
    xb = xs_ref[...]
    w1 = w1_ref[0]
    w2 = w2_ref[0]
    h = lax.dot_general(xb.astype(jnp.bfloat16), w1, (((1,), (1,)), ((), ())),
                        preferred_element_type=jnp.float32)
    mu = jnp.mean(h, axis=-1, keepdims=True)
    var = jnp.mean((h - mu) ** 2, axis=-1, keepdims=True)
    hn = (h - mu) * lax.rsqrt(var + 1e-5)
    hn = hn * lnw_ref[0] + lnb_ref[0]
    a = hn * 0.5 * (1.0 + lax.erf(hn * 0.7071067811865476))
    y = lax.dot_general(a.astype(jnp.bfloat16), w2, (((1,), (1,)), ((), ())),
                        preferred_element_type=jnp.float32)
    ys_ref[...] = y * ws_ref[:, :1]


def _ffn(xs, ws, W1, W2, lnw3, lnb3, be):
    grid_spec = pltpu.PrefetchScalarGridSpec(
        num_scalar_prefetch=1,
        grid=(_NB,),
        in_specs=[
            pl.BlockSpec((_BS, _D), lambda b, be: (b, 0)),
            pl.BlockSpec((_BS, _WL), lambda b, be: (b, 0)),
            pl.BlockSpec((1, _H, _D), lambda b, be: (be[b], 0, 0)),
            pl.BlockSpec((1, _D, _H), lambda b, be: (be[b], 0, 0)),
            pl.BlockSpec((1, 1, _H), lambda b, be: (be[b], 0, 0)),
            pl.BlockSpec((1, 1, _H), lambda b, be: (be[b], 0, 0)),
        ],
        out_specs=pl.BlockSpec((_BS, _D), lambda b, be: (b, 0)),
    )
    return pl.pallas_call(
        _ffn_kernel,
        grid_spec=grid_spec,
        out_shape=jax.ShapeDtypeStruct((_MPAD, _D), jnp.float32),
    )(be, xs, ws, W1, W2, lnw3, lnb3)


def _combine(ys, pos0f, pos1f):
    mesh = plsc.VectorSubcoreMesh(core_axis_name="c", subcore_axis_name="s")
    nchk = _TPW // _TCH

    @functools.partial(
        pl.kernel, mesh=mesh,
        out_type=jax.ShapeDtypeStruct((_N, _D), jnp.float32),
        scratch_types=[
            pltpu.VMEM((2, _TCH), jnp.int32),
            pltpu.VMEM((2, _TCH), jnp.int32),
            pltpu.VMEM((2, _TCH, _D), jnp.float32),
            pltpu.VMEM((2, _TCH, _D), jnp.float32),
            pltpu.VMEM((_TCH, _D), jnp.float32),
            pltpu.SemaphoreType.DMA,
            pltpu.SemaphoreType.DMA,
        ],
    )
    def combine_k(ys_hbm, pos0_hbm, pos1_hbm, out_hbm,
                  i0_v, i1_v, y0_v, y1_v, o_v, sem_a, sem_b):
        wid = lax.axis_index("s") * _NC + lax.axis_index("c")
        sems = (sem_a, sem_b)

        def issue(c):
            b = c % 2
            base = wid * _TPW + c * _TCH
            pltpu.sync_copy(pos0_hbm.at[pl.ds(base, _TCH)], i0_v.at[b])
            pltpu.sync_copy(pos1_hbm.at[pl.ds(base, _TCH)], i1_v.at[b])
            cp0 = pltpu.async_copy(ys_hbm.at[i0_v.at[b]], y0_v.at[b], sems[b])
            cp1 = pltpu.async_copy(ys_hbm.at[i1_v.at[b]], y1_v.at[b], sems[b])
            return cp0, cp1

        pend = issue(0)
        for c in range(nchk):
            b = c % 2
            base = wid * _TPW + c * _TCH
            pend[0].wait()
            pend[1].wait()
            if c + 1 < nchk:
                pend = issue(c + 1)

            @pl.loop(0, _D // _L)
            def _body(l, b=b):
                seg = pl.ds(l * _L, _L)
                for j in range(_TCH):
                    o_v[j, seg] = y0_v[b, j, seg] + y1_v[b, j, seg]

            pltpu.sync_copy(o_v, out_hbm.at[pl.ds(base, _TCH)])

    return combine_k(ys, pos0f, pos1f)


def kernel(x, Wg, W1, W2, ln_w, ln_b):
    pos0, pos1, w0s, w1s, be = _route(x, Wg)
    pos0f = pos0.reshape(_N)
    pos1f = pos1.reshape(_N)
    xs, ws = _dispatch(x, pos0f, pos1f, w0s, w1s)
    ys = _ffn(xs, ws, W1.astype(jnp.bfloat16), W2.astype(jnp.bfloat16),
              ln_w.reshape(_E, 1, _H), ln_b.reshape(_E, 1, _H),
              be.reshape(128)[:_NB])
    return _combine(ys, pos0f, pos1f)


# speedup vs baseline: 1.0727x; 1.0727x over previous
import functools
import jax
import jax.numpy as jnp
from jax import lax
from jax.experimental import pallas as pl
from jax.experimental.pallas import tpu as pltpu
from jax.experimental.pallas import tpu_sc as plsc

_N, _D, _H, _E = 2048, 1024, 512, 8
_BS = 128
_MPAD = 4096 + _E * _BS
_NB = _MPAD // _BS
_NCH = 16
_CH = _N // _NCH

_NC, _NS = 2, 16
_NW = _NC * _NS
_TPW = _N // _NW
_TCH = 16
_L = 16
_WL = 128


def _route_kernel(x_ref, wg_ref, pos0_ref, pos1_ref, w0_ref, w1_ref, be_ref):
    x = x_ref[...]
    wg = wg_ref[...]
    logits = lax.dot_general(x, wg, (((1,), (1,)), ((), ())),
                             preferred_element_type=jnp.float32)
    m = jnp.max(logits, axis=-1, keepdims=True)
    p = jnp.exp(logits - m)
    lane = lax.broadcasted_iota(jnp.int32, p.shape, 1)
    p0 = jnp.max(p, axis=-1, keepdims=True)
    e0 = jnp.min(jnp.where(p == p0, lane, _E), axis=-1, keepdims=True)
    p_m = jnp.where(lane == e0, -jnp.inf, p)
    p1 = jnp.max(p_m, axis=-1, keepdims=True)
    e1 = jnp.min(jnp.where(p_m == p1, lane, _E), axis=-1, keepdims=True)
    s = p0 + p1
    w0_ref[...] = jnp.broadcast_to(p0 / s, (_N, _WL))
    w1_ref[...] = jnp.broadcast_to(p1 / s, (_N, _WL))

    oh0 = (lane == e0).astype(jnp.float32)
    oh1 = (lane == e1).astype(jnp.float32)
    cnt = oh0 + oh1

    a = jnp.concatenate([cnt[c * _CH:(c + 1) * _CH, :] for c in range(_NCH)],
                        axis=1)
    ri = lax.broadcasted_iota(jnp.int32, (_CH, _CH), 0)
    ci = lax.broadcasted_iota(jnp.int32, (_CH, _CH), 1)
    ltri = (ri >= ci).astype(jnp.float32)
    b = lax.dot_general(ltri, a, (((1,), (0,)), ((), ())),
                        preferred_element_type=jnp.float32)
    tot = b[_CH - 1:_CH, :]
    same_e = (ri % _E) == (ci % _E)
    before = (ri // _E) < (ci // _E)
    mcarry = jnp.where(jnp.logical_and(same_e, before), 1.0, 0.0)
    carry = lax.dot_general(tot, mcarry, (((1,), (0,)), ((), ())),
                            preferred_element_type=jnp.float32)
    bc = b + carry
    csum = jnp.concatenate([bc[:, c * _E:(c + 1) * _E] for c in range(_NCH)],
                           axis=0)

    cnt_tot = csum[_N - 1:_N, :]
    cnt_pad = jnp.floor((cnt_tot + (_BS - 1)) * (1.0 / _BS)) * _BS
    ei = lax.broadcasted_iota(jnp.int32, (_E, _E), 0)
    ej = lax.broadcasted_iota(jnp.int32, (_E, _E), 1)
    excl = jnp.where(ei < ej, 1.0, 0.0)
    off = lax.dot_general(cnt_pad, excl, (((1,), (0,)), ((), ())),
                          preferred_element_type=jnp.float32)

    off_b = jnp.broadcast_to(off, (_N, _E))
    rank0 = jnp.sum(jnp.where(lane == e0, csum, 0.0), -1, keepdims=True) - 1.0
    rank1 = jnp.sum(jnp.where(lane == e1, csum, 0.0), -1, keepdims=True) - 1.0
    base0 = jnp.sum(jnp.where(lane == e0, off_b, 0.0), -1, keepdims=True)
    base1 = jnp.sum(jnp.where(lane == e1, off_b, 0.0), -1, keepdims=True)
    pos0_ref[...] = (base0 + rank0).astype(jnp.int32)
    pos1_ref[...] = (base1 + rank1).astype(jnp.int32)

    blk = lax.broadcasted_iota(jnp.int32, (1, 128), 1).astype(jnp.float32) * float(_BS)
    be = jnp.zeros((1, 128), jnp.float32)
    for e in range(_E):
        lo = off[:, e:e + 1]
        hi = lo + cnt_pad[:, e:e + 1]
        ind = jnp.where(jnp.logical_and(blk >= lo, blk < hi), 1.0, 0.0)
        be = be + float(e) * ind
    be_ref[...] = be.astype(jnp.int32)


def _route(x, Wg):
    return pl.pallas_call(
        _route_kernel,
        out_shape=[
            jax.ShapeDtypeStruct((_N, 1), jnp.int32),
            jax.ShapeDtypeStruct((_N, 1), jnp.int32),
            jax.ShapeDtypeStruct((_N, _WL), jnp.float32),
            jax.ShapeDtypeStruct((_N, _WL), jnp.float32),
            jax.ShapeDtypeStruct((1, 128), jnp.int32),
        ],
    )(x, Wg)


def _dispatch(xb, pos0f, pos1f, w0s, w1s):
    mesh = plsc.VectorSubcoreMesh(core_axis_name="c", subcore_axis_name="s")

    @functools.partial(
        pl.kernel, mesh=mesh,
        out_type=[
            jax.ShapeDtypeStruct((_MPAD, _D), jnp.float32),
            jax.ShapeDtypeStruct((_MPAD, _WL), jnp.float32),
        ],
        scratch_types=[
            pltpu.VMEM((_TPW,), jnp.int32),
            pltpu.VMEM((_TPW,), jnp.int32),
            pltpu.VMEM((_TPW, _D), jnp.float32),
            pltpu.VMEM((_TPW, _WL), jnp.float32),
            pltpu.VMEM((_TPW, _WL), jnp.float32),
            pltpu.SemaphoreType.DMA,
            pltpu.SemaphoreType.DMA,
        ],
    )
    def dispatch_k(x_hbm, pos0_hbm, pos1_hbm, w0_hbm, w1_hbm, xs_hbm, ws_hbm,
                   i0_v, i1_v, rows_v, w0_v, w1_v, sem, semw):
        wid = lax.axis_index("s") * _NC + lax.axis_index("c")
        base = wid * _TPW
        pltpu.sync_copy(pos0_hbm.at[pl.ds(base, _TPW)], i0_v)
        pltpu.sync_copy(pos1_hbm.at[pl.ds(base, _TPW)], i1_v)
        pltpu.sync_copy(x_hbm.at[pl.ds(base, _TPW)], rows_v)
        pltpu.sync_copy(w0_hbm.at[pl.ds(base, _TPW)], w0_v)
        pltpu.sync_copy(w1_hbm.at[pl.ds(base, _TPW)], w1_v)
        cw0 = pltpu.async_copy(w0_v, ws_hbm.at[i0_v], semw)
        cw1 = pltpu.async_copy(w1_v, ws_hbm.at[i1_v], semw)
        cx0 = pltpu.async_copy(rows_v, xs_hbm.at[i0_v], sem)
        cx1 = pltpu.async_copy(rows_v, xs_hbm.at[i1_v], sem)
        cw0.wait()
        cw1.wait()
        cx0.wait()
        cx1.wait()

    return dispatch_k(xb, pos0f, pos1f, w0s, w1s)


def _ffn_kernel(be_ref, xs_ref, ws_ref, w1_ref, w2_ref, lnw_ref, lnb_ref,
                ys_ref):
    xb = xs_ref[...]
    w1 = w1_ref[0]
    w2 = w2_ref[0]
    h = lax.dot_general(xb, w1, (((1,), (1,)), ((), ())),
                        preferred_element_type=jnp.float32)
    mu = jnp.mean(h, axis=-1, keepdims=True)
    var = jnp.mean((h - mu) ** 2, axis=-1, keepdims=True)
    hn = (h - mu) * lax.rsqrt(var + 1e-5)
    hn = hn * lnw_ref[0] + lnb_ref[0]
    a = hn * 0.5 * (1.0 + lax.erf(hn * 0.7071067811865476))
    y = lax.dot_general(a, w2, (((1,), (1,)), ((), ())),
                        preferred_element_type=jnp.float32)
    ys_ref[...] = y * ws_ref[:, :1]


def _ffn(xs, ws, W1, W2, lnw3, lnb3, be):
    grid_spec = pltpu.PrefetchScalarGridSpec(
        num_scalar_prefetch=1,
        grid=(_NB,),
        in_specs=[
            pl.BlockSpec((_BS, _D), lambda b, be: (b, 0)),
            pl.BlockSpec((_BS, _WL), lambda b, be: (b, 0)),
            pl.BlockSpec((1, _H, _D), lambda b, be: (be[b], 0, 0)),
            pl.BlockSpec((1, _D, _H), lambda b, be: (be[b], 0, 0)),
            pl.BlockSpec((1, 1, _H), lambda b, be: (be[b], 0, 0)),
            pl.BlockSpec((1, 1, _H), lambda b, be: (be[b], 0, 0)),
        ],
        out_specs=pl.BlockSpec((_BS, _D), lambda b, be: (b, 0)),
    )
    return pl.pallas_call(
        _ffn_kernel,
        grid_spec=grid_spec,
        out_shape=jax.ShapeDtypeStruct((_MPAD, _D), jnp.float32),
    )(be, xs, ws, W1, W2, lnw3, lnb3)


def _combine(ys, pos0f, pos1f):
    mesh = plsc.VectorSubcoreMesh(core_axis_name="c", subcore_axis_name="s")
    nchk = _TPW // _TCH

    @functools.partial(
        pl.kernel, mesh=mesh,
        out_type=jax.ShapeDtypeStruct((_N, _D), jnp.float32),
        scratch_types=[
            pltpu.VMEM((2, _TCH), jnp.int32),
            pltpu.VMEM((2, _TCH), jnp.int32),
            pltpu.VMEM((2, _TCH, _D), jnp.float32),
            pltpu.VMEM((2, _TCH, _D), jnp.float32),
            pltpu.VMEM((_TCH, _D), jnp.float32),
            pltpu.SemaphoreType.DMA,
            pltpu.SemaphoreType.DMA,
        ],
    )
    def combine_k(ys_hbm, pos0_hbm, pos1_hbm, out_hbm,
                  i0_v, i1_v, y0_v, y1_v, o_v, sem_a, sem_b):
        wid = lax.axis_index("s") * _NC + lax.axis_index("c")
        sems = (sem_a, sem_b)

        def issue(c):
            b = c % 2
            base = wid * _TPW + c * _TCH
            pltpu.sync_copy(pos0_hbm.at[pl.ds(base, _TCH)], i0_v.at[b])
            pltpu.sync_copy(pos1_hbm.at[pl.ds(base, _TCH)], i1_v.at[b])
            cp0 = pltpu.async_copy(ys_hbm.at[i0_v.at[b]], y0_v.at[b], sems[b])
            cp1 = pltpu.async_copy(ys_hbm.at[i1_v.at[b]], y1_v.at[b], sems[b])
            return cp0, cp1

        pend = issue(0)
        for c in range(nchk):
            b = c % 2
            base = wid * _TPW + c * _TCH
            pend[0].wait()
            pend[1].wait()
            if c + 1 < nchk:
                pend = issue(c + 1)

            @pl.loop(0, _D // _L)
            def _body(l, b=b):
                seg = pl.ds(l * _L, _L)
                for j in range(_TCH):
                    o_v[j, seg] = y0_v[b, j, seg] + y1_v[b, j, seg]

            pltpu.sync_copy(o_v, out_hbm.at[pl.ds(base, _TCH)])

    return combine_k(ys, pos0f, pos1f)


def kernel(x, Wg, W1, W2, ln_w, ln_b):
    pos0, pos1, w0s, w1s, be = _route(x, Wg)
    pos0f = pos0.reshape(_N)
    pos1f = pos1.reshape(_N)
    xs, ws = _dispatch(x, pos0f, pos1f, w0s, w1s)
    ys = _ffn(xs, ws, W1, W2, ln_w.reshape(_E, 1, _H), ln_b.reshape(_E, 1, _H),
              be.reshape(128)[:_NB])
    return _combine(ys, pos0f, pos1f)
